# baseline (speedup 1.0000x reference)
import jax
import jax.numpy as jnp
from jax import lax
from jax.experimental import pallas as pl
from jax.experimental.pallas import tpu as pltpu
from jax.experimental.pallas import tpu_sc as plsc

_NUM_FRAMES = 100000
_B = 16384
_D_BE = 16
_P_GO, _P_BP, _P_TR, _P_EX = 8, 64, 8, 16

_NC, _NS = 2, 16
_NW = _NC * _NS
_BPW = _B // _NW
_CH = 128
_NCH = _BPW // _CH

_mesh = plsc.VectorSubcoreMesh(core_axis_name="c", subcore_axis_name="s")


def _body(ids_hbm, be_hbm, go_hbm, bp_hbm, tr_hbm, ex_hbm,
          be_out, go_out, bp_out, tr_out, ex_out,
          idx_v, be_v, go_v, bp_v, tr_v, ex_v, sem):
    wid = lax.axis_index("s") * _NC + lax.axis_index("c")
    base = wid * _BPW

    pltpu.sync_copy(ids_hbm.at[pl.ds(wid * _NCH, _NCH)], idx_v)

    copies = []
    for tbl, dst in ((go_hbm, go_v), (bp_hbm, bp_v),
                     (tr_hbm, tr_v), (ex_hbm, ex_v)):
        for j in range(_NCH):
            copies.append(
                pltpu.async_copy(tbl.at[idx_v.at[j]],
                                 dst.at[pl.ds(j * _CH, _CH)], sem))

    pltpu.sync_copy(be_hbm, be_v.at[pl.ds(0, 1)])
    row = be_v[0, :]

    def _fill(i, _):
        be_v[i, :] = row
        return 0

    lax.fori_loop(1, _BPW, _fill, 0)
    pltpu.sync_copy(be_v, be_out.at[pl.ds(base, _BPW)])

    for c in copies:
        c.wait()
    pltpu.sync_copy(go_v, go_out.at[pl.ds(base, _BPW)])
    pltpu.sync_copy(bp_v, bp_out.at[pl.ds(base, _BPW)])
    pltpu.sync_copy(tr_v, tr_out.at[pl.ds(base, _BPW)])
    pltpu.sync_copy(ex_v, ex_out.at[pl.ds(base, _BPW)])


def _build(interpret=False):
    return pl.kernel(
        _body,
        mesh=_mesh,
        compiler_params=pltpu.CompilerParams(use_tc_tiling_on_sc=False),
        out_type=[
            jax.ShapeDtypeStruct((_B, _D_BE), jnp.float32),
            jax.ShapeDtypeStruct((_B, _P_GO), jnp.float32),
            jax.ShapeDtypeStruct((_B, _P_BP), jnp.float32),
            jax.ShapeDtypeStruct((_B, _P_TR), jnp.float32),
            jax.ShapeDtypeStruct((_B, _P_EX), jnp.float32),
        ],
        scratch_types=[
            pltpu.VMEM((_NCH, _CH), jnp.int32),
            pltpu.VMEM((_BPW, _D_BE), jnp.float32),
            pltpu.VMEM((_BPW, _P_GO), jnp.float32),
            pltpu.VMEM((_BPW, _P_BP), jnp.float32),
            pltpu.VMEM((_BPW, _P_TR), jnp.float32),
            pltpu.VMEM((_BPW, _P_EX), jnp.float32),
            pltpu.SemaphoreType.DMA,
        ],
        interpret=interpret,
    )


_gather_all = _build()


def kernel(frame_ids, betas_w, global_orient_w, body_pose_w, transl_w,
           expression_w):
    ids2d = frame_ids.astype(jnp.int32).reshape(_B // _CH, _CH)
    go_p = jnp.pad(global_orient_w, ((0, 0), (0, _P_GO - 3)))
    bp_p = jnp.pad(body_pose_w, ((0, 0), (0, _P_BP - 63)))
    tr_p = jnp.pad(transl_w, ((0, 0), (0, _P_TR - 3)))
    ex_p = jnp.pad(expression_w, ((0, 0), (0, _P_EX - 10)))
    be, go, bp, tr, ex = _gather_all(ids2d, betas_w, go_p, bp_p, tr_p, ex_p)
    return (be, go[:, :3], bp[:, :63], tr[:, :3], ex[:, :10])


# speedup vs baseline: 1.0387x; 1.0387x over previous
import jax
import jax.numpy as jnp
from jax import lax
from jax.experimental import pallas as pl
from jax.experimental.pallas import tpu as pltpu
from jax.experimental.pallas import tpu_sc as plsc

_F = 100000
_B = 16384
_D_GO, _D_BP, _D_TR, _D_EX, _D_BE = 3, 63, 3, 10, 16
_R_GO, _R_BP, _R_TR, _R_EX = 8, 64, 8, 16

_NW = 32
_BPW = _B // _NW
_NCK = _BPW // 16

_mesh = plsc.VectorSubcoreMesh(core_axis_name="c", subcore_axis_name="s")


def _body(ids_hbm, be_hbm, go_hbm, bp_hbm, tr_hbm, ex_hbm,
          be_t, go_t, bp_t, tr_t, ex_t,
          idx_v, be_row, go_v, bp_v, tr_v, ex_v,
          be_p, go_p, bp_p, tr_p, ex_p, sem, wsem):
    wid = lax.axis_index("s") * 2 + lax.axis_index("c")
    base = wid * _BPW

    pltpu.sync_copy(ids_hbm.at[pl.ds(base, _BPW)], idx_v)

    def _row(i, _):
        vec = idx_v[pl.ds(i * 16, 16)]
        for l in range(16):
            idx = vec[l]
            r = i * 16 + l
            pltpu.make_async_copy(go_hbm.at[pl.ds(idx, 1)],
                                  go_v.at[pl.ds(r, 1)], sem).start()
            pltpu.make_async_copy(bp_hbm.at[pl.ds(idx, 1)],
                                  bp_v.at[pl.ds(r, 1)], sem).start()
            pltpu.make_async_copy(tr_hbm.at[pl.ds(idx, 1)],
                                  tr_v.at[pl.ds(r, 1)], sem).start()
            pltpu.make_async_copy(ex_hbm.at[pl.ds(idx, 1)],
                                  ex_v.at[pl.ds(r, 1)], sem).start()
        return 0

    lax.fori_loop(0, _NCK, _row, 0)

    pltpu.sync_copy(be_hbm, be_row)
    vec = be_row[0, :]
    lanes = lax.iota(jnp.int32, 16)
    for d in range(_D_BE):
        val = jnp.sum(jnp.where(lanes == d, vec, 0.0))
        bvec = jnp.full((16,), val, jnp.float32)

        def _fill(i, _):
            be_p[d, pl.ds(i * 16, 16)] = bvec
            return 0

        lax.fori_loop(0, _NCK, _fill, 0)
    w_be = pltpu.async_copy(be_p, be_t.at[:, pl.ds(base, _BPW)], wsem)

    pltpu.make_async_copy(go_hbm.at[pl.ds(0, _BPW)], go_v, sem).wait()
    pltpu.make_async_copy(bp_hbm.at[pl.ds(0, _BPW)], bp_v, sem).wait()
    pltpu.make_async_copy(tr_hbm.at[pl.ds(0, _BPW)], tr_v, sem).wait()
    pltpu.make_async_copy(ex_hbm.at[pl.ds(0, _BPW)], ex_v, sem).wait()

    iota = lax.iota(jnp.int32, 16)
    writes = [w_be]
    for src, dst, out, d in ((go_v, go_p, go_t, _D_GO),
                             (bp_v, bp_p, bp_t, _D_BP),
                             (tr_v, tr_p, tr_t, _D_TR),
                             (ex_v, ex_p, ex_t, _D_EX)):
        def _tile(i, _, src=src, dst=dst, d=d):
            rows = i * 16 + iota
            for j in range(d):
                cols = jnp.full((16,), j, jnp.int32)
                dst[j, pl.ds(i * 16, 16)] = plsc.load_gather(src, [rows, cols])
            return 0

        lax.fori_loop(0, _NCK, _tile, 0)
        writes.append(pltpu.async_copy(dst, out.at[:, pl.ds(base, _BPW)],
                                       wsem))
    for w in writes:
        w.wait()


def _build(interpret=False):
    return pl.kernel(
        _body,
        mesh=_mesh,
        compiler_params=pltpu.CompilerParams(use_tc_tiling_on_sc=False,
                                             needs_layout_passes=False),
        out_type=[
            jax.ShapeDtypeStruct((_D_BE, _B), jnp.float32),
            jax.ShapeDtypeStruct((_R_GO, _B), jnp.float32),
            jax.ShapeDtypeStruct((_R_BP, _B), jnp.float32),
            jax.ShapeDtypeStruct((_R_TR, _B), jnp.float32),
            jax.ShapeDtypeStruct((_R_EX, _B), jnp.float32),
        ],
        scratch_types=[
            pltpu.VMEM((_BPW,), jnp.int32),
            pltpu.VMEM((1, _D_BE), jnp.float32),
            pltpu.VMEM((_BPW, _D_GO), jnp.float32),
            pltpu.VMEM((_BPW, _D_BP), jnp.float32),
            pltpu.VMEM((_BPW, _D_TR), jnp.float32),
            pltpu.VMEM((_BPW, _D_EX), jnp.float32),
            pltpu.VMEM((_D_BE, _BPW), jnp.float32),
            pltpu.VMEM((_R_GO, _BPW), jnp.float32),
            pltpu.VMEM((_R_BP, _BPW), jnp.float32),
            pltpu.VMEM((_R_TR, _BPW), jnp.float32),
            pltpu.VMEM((_R_EX, _BPW), jnp.float32),
            pltpu.SemaphoreType.DMA,
            pltpu.SemaphoreType.DMA,
        ],
        interpret=interpret,
    )


_gather_all = _build()


def kernel(frame_ids, betas_w, global_orient_w, body_pose_w, transl_w,
           expression_w):
    ids = frame_ids.astype(jnp.int32)
    be_t, go_t, bp_t, tr_t, ex_t = _gather_all(
        ids, betas_w, global_orient_w, body_pose_w, transl_w, expression_w)
    return (be_t.T, go_t[:_D_GO].T, bp_t[:_D_BP].T, tr_t[:_D_TR].T,
            ex_t[:_D_EX].T)


# speedup vs baseline: 1.8438x; 1.7750x over previous
import jax
import jax.numpy as jnp
from jax import lax
from jax.experimental import pallas as pl
from jax.experimental.pallas import tpu as pltpu
from jax.experimental.pallas import tpu_sc as plsc

_F = 100000
_B = 16384
_W = 128
_D_GO, _D_BP, _D_TR, _D_EX, _D_BE = 3, 63, 3, 10, 16
_C_GO, _C_TR, _C_EX, _C_BP = 0, 3, 6, 16
_R_GO, _R_BP, _R_TR, _R_EX = 8, 64, 8, 16

_NW = 32
_BPW = _B // _NW
_CH = 128
_NCH = _BPW // _CH
_NCK = _BPW // 16

_mesh = plsc.VectorSubcoreMesh(core_axis_name="c", subcore_axis_name="s")


def _body(ids_hbm, be_hbm, tbl_hbm,
          be_t, go_t, bp_t, tr_t, ex_t,
          idx_v, be_row, rows_v, be_p, go_p, bp_p, tr_p, ex_p, sem, wsem):
    wid = lax.axis_index("s") * 2 + lax.axis_index("c")
    base = wid * _BPW

    pltpu.sync_copy(ids_hbm.at[pl.ds(wid * _NCH, _NCH)], idx_v)
    gathers = [pltpu.async_copy(tbl_hbm.at[idx_v.at[j]],
                                rows_v.at[pl.ds(j * _CH, _CH)], sem)
               for j in range(_NCH)]

    pltpu.sync_copy(be_hbm, be_row)
    vec = be_row[0, :]
    lanes = lax.iota(jnp.int32, 16)
    for d in range(_D_BE):
        val = jnp.sum(jnp.where(lanes == d, vec, 0.0))
        bvec = jnp.full((16,), val, jnp.float32)

        def _fill(i, _, d=d, bvec=bvec):
            be_p[d, pl.ds(i * 16, 16)] = bvec
            return 0

        lax.fori_loop(0, _NCK, _fill, 0)
    writes = [pltpu.async_copy(be_p, be_t.at[:, pl.ds(base, _BPW)], wsem)]

    for g in gathers:
        g.wait()

    iota = lax.iota(jnp.int32, 16)
    for dst, out, c0, d in ((go_p, go_t, _C_GO, _D_GO),
                            (tr_p, tr_t, _C_TR, _D_TR),
                            (ex_p, ex_t, _C_EX, _D_EX),
                            (bp_p, bp_t, _C_BP, _D_BP)):
        def _tile(i, _, dst=dst, c0=c0, d=d):
            rows = i * 16 + iota
            for j in range(d):
                cols = jnp.full((16,), c0 + j, jnp.int32)
                dst[j, pl.ds(i * 16, 16)] = plsc.load_gather(
                    rows_v, [rows, cols])
            return 0

        lax.fori_loop(0, _NCK, _tile, 0)
        writes.append(pltpu.async_copy(dst, out.at[:, pl.ds(base, _BPW)],
                                       wsem))
    for w in writes:
        w.wait()


def _build(interpret=False):
    return pl.kernel(
        _body,
        mesh=_mesh,
        compiler_params=pltpu.CompilerParams(use_tc_tiling_on_sc=False,
                                             needs_layout_passes=False),
        out_type=[
            jax.ShapeDtypeStruct((_D_BE, _B), jnp.float32),
            jax.ShapeDtypeStruct((_R_GO, _B), jnp.float32),
            jax.ShapeDtypeStruct((_R_BP, _B), jnp.float32),
            jax.ShapeDtypeStruct((_R_TR, _B), jnp.float32),
            jax.ShapeDtypeStruct((_R_EX, _B), jnp.float32),
        ],
        scratch_types=[
            pltpu.VMEM((_NCH, _CH), jnp.int32),
            pltpu.VMEM((1, _D_BE), jnp.float32),
            pltpu.VMEM((_BPW, _W), jnp.float32),
            pltpu.VMEM((_D_BE, _BPW), jnp.float32),
            pltpu.VMEM((_R_GO, _BPW), jnp.float32),
            pltpu.VMEM((_R_BP, _BPW), jnp.float32),
            pltpu.VMEM((_R_TR, _BPW), jnp.float32),
            pltpu.VMEM((_R_EX, _BPW), jnp.float32),
            pltpu.SemaphoreType.DMA,
            pltpu.SemaphoreType.DMA,
        ],
        interpret=interpret,
    )


_gather_all = _build()


def kernel(frame_ids, betas_w, global_orient_w, body_pose_w, transl_w,
           expression_w):
    ids2d = frame_ids.astype(jnp.int32).reshape(_B // _CH, _CH)
    merged = jnp.pad(
        jnp.concatenate([global_orient_w, transl_w, expression_w,
                         body_pose_w], axis=1),
        ((0, 0), (0, _W - _C_BP - _D_BP)))
    be_t, go_t, bp_t, tr_t, ex_t = _gather_all(ids2d, betas_w, merged)
    return (be_t.T, go_t[:_D_GO].T, bp_t[:_D_BP].T, tr_t[:_D_TR].T,
            ex_t[:_D_EX].T)
